# Initial kernel scaffold; baseline (speedup 1.0000x reference)
import jax
import jax.numpy as jnp
from jax import lax
from jax.experimental import pallas as pl
from jax.experimental.pallas import tpu as pltpu
from jax.experimental.pallas import tpu_sc as plsc

N = 10000
E = 320000
NC = 2
NS = 16
CHUNK = 128
NPAD = 10240
EPAD = 327680
NCHUNK = EPAD // (NC * NS) // CHUNK
ROWS_PER_TILE = NPAD // NS


def _make_sc_agg(d, with_deg):
  mesh = plsc.VectorSubcoreMesh(core_axis_name="c", subcore_axis_name="s")
  out_type = [jax.ShapeDtypeStruct((NC, NPAD, d), jnp.float32)]
  scratch = [
      pltpu.VMEM((NCHUNK, CHUNK), jnp.int32),
      pltpu.VMEM((NCHUNK, CHUNK), jnp.int32),
      pltpu.VMEM((CHUNK, d), jnp.float32),
      pltpu.VMEM((CHUNK, d), jnp.float32),
      pltpu.VMEM((64, d), jnp.float32),
      pltpu.VMEM_SHARED((NPAD, d), jnp.float32),
      pltpu.SemaphoreType.DMA,
      pltpu.SemaphoreType.DMA,
  ]
  if with_deg:
    out_type.append(jax.ShapeDtypeStruct((NC, NPAD), jnp.float32))
    scratch += [
        pltpu.VMEM((CHUNK,), jnp.float32),
        pltpu.VMEM((ROWS_PER_TILE,), jnp.float32),
        pltpu.VMEM_SHARED((NPAD,), jnp.float32),
    ]

  def body(hw_hbm, src_hbm, dst_hbm, agg_out, *rest):
    if with_deg:
      (deg_out, src_v, dst_v, r0, r1, zbuf, agg_sh, sem0, sem1,
       ones_v, zdeg, deg_sh) = rest
    else:
      (src_v, dst_v, r0, r1, zbuf, agg_sh, sem0, sem1) = rest

    c = lax.axis_index("c")
    s = lax.axis_index("s")
    wid = c * NS + s
    base = s * ROWS_PER_TILE
    zeros16 = jnp.zeros((16,), jnp.float32)

    @pl.loop(0, 64)
    def _zero_rows(r):
      for k in range(d // 16):
        zbuf[r, pl.ds(k * 16, 16)] = zeros16

    if with_deg:
      ones16 = jnp.ones((16,), jnp.float32)
      for k in range(CHUNK // 16):
        ones_v[pl.ds(k * 16, 16)] = ones16

      @pl.loop(0, ROWS_PER_TILE // 16)
      def _zero_deg(r):
        zdeg[pl.ds(r * 16, 16)] = zeros16

    pltpu.sync_copy(src_hbm.at[pl.ds(wid * NCHUNK, NCHUNK)], src_v)
    pltpu.sync_copy(dst_hbm.at[pl.ds(wid * NCHUNK, NCHUNK)], dst_v)
    for k in range(ROWS_PER_TILE // 64):
      pltpu.sync_copy(zbuf, agg_sh.at[pl.ds(base + k * 64, 64)])
    if with_deg:
      pltpu.sync_copy(zdeg, deg_sh.at[pl.ds(base, ROWS_PER_TILE)])
    plsc.subcore_barrier()

    rows = (r0, r1)
    sems = (sem0, sem1)
    pltpu.async_copy(hw_hbm.at[src_v.at[0]], r0, sem0)
    pltpu.async_copy(hw_hbm.at[src_v.at[1]], r1, sem1)

    @pl.loop(0, NCHUNK // 2 - 1)
    def _chunks(g):
      j = g * 2
      for b in range(2):
        jj = j + b
        pltpu.make_async_copy(hw_hbm.at[src_v.at[jj]], rows[b], sems[b]).wait()
        pltpu.sync_copy(rows[b], agg_sh.at[dst_v.at[jj]], add=True)
        if with_deg:
          pltpu.sync_copy(ones_v, deg_sh.at[dst_v.at[jj]], add=True)
        pltpu.async_copy(hw_hbm.at[src_v.at[jj + 2]], rows[b], sems[b])

    for b in range(2):
      jj = NCHUNK - 2 + b
      pltpu.make_async_copy(hw_hbm.at[src_v.at[jj]], rows[b], sems[b]).wait()
      pltpu.sync_copy(rows[b], agg_sh.at[dst_v.at[jj]], add=True)
      if with_deg:
        pltpu.sync_copy(ones_v, deg_sh.at[dst_v.at[jj]], add=True)

    plsc.subcore_barrier()
    pltpu.sync_copy(agg_sh.at[pl.ds(base, ROWS_PER_TILE)],
                    agg_out.at[c, pl.ds(base, ROWS_PER_TILE)])
    if with_deg:
      pltpu.sync_copy(deg_sh.at[pl.ds(base, ROWS_PER_TILE)],
                      deg_out.at[c, pl.ds(base, ROWS_PER_TILE)])

  return pl.kernel(body, out_type=out_type, mesh=mesh, scratch_types=scratch)


_BN = 1000


def _tc_first(x, wcatT, b, dout):
  din = x.shape[1]

  def body(x_ref, w_ref, b_ref, hs_ref, hw_ref):
    xw = jnp.dot(x_ref[...], w_ref[...], preferred_element_type=jnp.float32)
    hs_ref[...] = xw[:, :dout] + b_ref[...]
    hw_ref[...] = xw[:, dout:]

  return pl.pallas_call(
      body,
      grid=(N // _BN,),
      in_specs=[
          pl.BlockSpec((_BN, din), lambda i: (i, 0)),
          pl.BlockSpec((din, 2 * dout), lambda i: (0, 0)),
          pl.BlockSpec((1, dout), lambda i: (0, 0)),
      ],
      out_specs=[
          pl.BlockSpec((_BN, dout), lambda i: (i, 0)),
          pl.BlockSpec((_BN, dout), lambda i: (i, 0)),
      ],
      out_shape=[jax.ShapeDtypeStruct((N, dout), jnp.float32)] * 2,
  )(x, wcatT, b)


def _tc_mid(hs_prev, agg, deg3, wcatT, b, dout):
  din = hs_prev.shape[1]

  def body(hs_ref, agg_ref, deg_ref, w_ref, b_ref, hs_out, hw_out):
    a = agg_ref[0] + agg_ref[1]
    dg = deg_ref[0] + deg_ref[1]
    inv = 1.0 / jnp.maximum(dg, 1.0)
    h = jnp.maximum(hs_ref[...] + a * inv, 0.0)
    hw = jnp.dot(h, w_ref[...], preferred_element_type=jnp.float32)
    hs_out[...] = hw[:, :dout] + b_ref[...]
    hw_out[...] = hw[:, dout:]

  return pl.pallas_call(
      body,
      grid=(N // _BN,),
      in_specs=[
          pl.BlockSpec((_BN, din), lambda i: (i, 0)),
          pl.BlockSpec((NC, _BN, din), lambda i: (0, i, 0)),
          pl.BlockSpec((NC, _BN, 1), lambda i: (0, i, 0)),
          pl.BlockSpec((din, 2 * dout), lambda i: (0, 0)),
          pl.BlockSpec((1, dout), lambda i: (0, 0)),
      ],
      out_specs=[
          pl.BlockSpec((_BN, dout), lambda i: (i, 0)),
          pl.BlockSpec((_BN, dout), lambda i: (i, 0)),
      ],
      out_shape=[jax.ShapeDtypeStruct((N, dout), jnp.float32)] * 2,
  )(hs_prev, agg, deg3, wcatT, b)


def _tc_final(hs2, agg, deg3):
  dout = hs2.shape[1]

  def body(hs_ref, agg_ref, deg_ref, out_ref):
    a = agg_ref[0] + agg_ref[1]
    dg = deg_ref[0] + deg_ref[1]
    inv = 1.0 / jnp.maximum(dg, 1.0)
    out_ref[...] = hs_ref[...] + a * inv

  return pl.pallas_call(
      body,
      grid=(N // _BN,),
      in_specs=[
          pl.BlockSpec((_BN, dout), lambda i: (i, 0)),
          pl.BlockSpec((NC, _BN, dout), lambda i: (0, i, 0)),
          pl.BlockSpec((NC, _BN, 1), lambda i: (0, i, 0)),
      ],
      out_specs=pl.BlockSpec((_BN, dout), lambda i: (i, 0)),
      out_shape=jax.ShapeDtypeStruct((N, dout), jnp.float32),
  )(hs2, agg, deg3)


@jax.jit
def kernel(x, edge_index, W_self0, W_neigh0, b0, W_self1, W_neigh1, b1,
           W_self2, W_neigh2, b2):
  src = edge_index[0]
  dst = edge_index[1]
  pad = EPAD - E
  src_p = jnp.concatenate(
      [src, jnp.zeros((pad,), jnp.int32)]).reshape(EPAD // CHUNK, CHUNK)
  dst_p = jnp.concatenate(
      [dst, jnp.full((pad,), N, jnp.int32)]).reshape(EPAD // CHUNK, CHUNK)

  w0 = jnp.concatenate([W_self0, W_neigh0], axis=0).T
  w1 = jnp.concatenate([W_self1, W_neigh1], axis=0).T
  w2 = jnp.concatenate([W_self2, W_neigh2], axis=0).T

  hs0, hw0 = _tc_first(x, w0, b0.reshape(1, -1), 128)
  agg0, deg = _make_sc_agg(128, True)(hw0, src_p, dst_p)
  deg3 = deg.reshape(NC, NPAD, 1)
  hs1, hw1 = _tc_mid(hs0, agg0, deg3, w1, b1.reshape(1, -1), 128)
  agg1 = _make_sc_agg(128, False)(hw1, src_p, dst_p)[0]
  hs2, hw2 = _tc_mid(hs1, agg1, deg3, w2, b2.reshape(1, -1), 64)
  agg2 = _make_sc_agg(64, False)(hw2, src_p, dst_p)[0]
  return _tc_final(hs2, agg2, deg3)


# speedup vs baseline: 6.0683x; 6.0683x over previous
import jax
import jax.numpy as jnp
from jax import lax
from jax.experimental import pallas as pl
from jax.experimental.pallas import tpu as pltpu
from jax.experimental.pallas import tpu_sc as plsc

N = 10000
E = 320000
NC = 2
NS = 16
CHUNK = 128
NPAD = 10240
EPAD = 327680
NCHUNK = EPAD // NS // CHUNK
ROWS_PER_TILE = NPAD // NS


def _make_sc_agg(dh, with_deg):
  mesh = plsc.VectorSubcoreMesh(core_axis_name="c", subcore_axis_name="s")
  out_type = [jax.ShapeDtypeStruct((NC, NPAD, dh), jnp.float32)]
  scratch = [
      pltpu.VMEM((NCHUNK, CHUNK), jnp.int32),
      pltpu.VMEM((NCHUNK, CHUNK), jnp.int32),
      pltpu.VMEM((CHUNK, dh), jnp.float32),
      pltpu.VMEM((CHUNK, dh), jnp.float32),
      pltpu.VMEM((64, dh), jnp.float32),
      pltpu.VMEM_SHARED((NPAD, dh), jnp.float32),
      pltpu.SemaphoreType.DMA,
      pltpu.SemaphoreType.DMA,
  ]
  if with_deg:
    out_type.append(jax.ShapeDtypeStruct((NC, NPAD), jnp.float32))
    scratch += [
        pltpu.VMEM((CHUNK,), jnp.float32),
        pltpu.VMEM((ROWS_PER_TILE,), jnp.float32),
        pltpu.VMEM_SHARED((NPAD,), jnp.float32),
    ]

  def body(hw_hbm, src_hbm, dst_hbm, agg_out, *rest):
    if with_deg:
      (deg_out, src_v, dst_v, r0, r1, zbuf, agg_sh, sem0, sem1,
       ones_v, zdeg, deg_sh) = rest
    else:
      (src_v, dst_v, r0, r1, zbuf, agg_sh, sem0, sem1) = rest

    c = lax.axis_index("c")
    s = lax.axis_index("s")
    base = s * ROWS_PER_TILE
    zeros16 = jnp.zeros((16,), jnp.float32)

    @pl.loop(0, 64)
    def _zero_rows(r):
      for k in range(dh // 16):
        zbuf[r, pl.ds(k * 16, 16)] = zeros16

    if with_deg:
      ones16 = jnp.ones((16,), jnp.float32)
      for k in range(CHUNK // 16):
        ones_v[pl.ds(k * 16, 16)] = ones16

      @pl.loop(0, ROWS_PER_TILE // 16)
      def _zero_deg(r):
        zdeg[pl.ds(r * 16, 16)] = zeros16

    pltpu.sync_copy(src_hbm.at[c, pl.ds(s * NCHUNK, NCHUNK)], src_v)
    pltpu.sync_copy(dst_hbm.at[pl.ds(s * NCHUNK, NCHUNK)], dst_v)
    for k in range(ROWS_PER_TILE // 64):
      pltpu.sync_copy(zbuf, agg_sh.at[pl.ds(base + k * 64, 64)])
    if with_deg:
      pltpu.sync_copy(zdeg, deg_sh.at[pl.ds(base, ROWS_PER_TILE)])
    plsc.subcore_barrier()

    rows = (r0, r1)
    sems = (sem0, sem1)
    pltpu.async_copy(hw_hbm.at[src_v.at[0]], r0, sem0)
    pltpu.async_copy(hw_hbm.at[src_v.at[1]], r1, sem1)

    @pl.loop(0, NCHUNK // 2 - 1)
    def _chunks(g):
      j = g * 2
      for b in range(2):
        jj = j + b
        pltpu.make_async_copy(hw_hbm.at[src_v.at[jj]], rows[b], sems[b]).wait()
        pltpu.sync_copy(rows[b], agg_sh.at[dst_v.at[jj]], add=True)
        if with_deg:
          pltpu.sync_copy(ones_v, deg_sh.at[dst_v.at[jj]], add=True)
        pltpu.async_copy(hw_hbm.at[src_v.at[jj + 2]], rows[b], sems[b])

    for b in range(2):
      jj = NCHUNK - 2 + b
      pltpu.make_async_copy(hw_hbm.at[src_v.at[jj]], rows[b], sems[b]).wait()
      pltpu.sync_copy(rows[b], agg_sh.at[dst_v.at[jj]], add=True)
      if with_deg:
        pltpu.sync_copy(ones_v, deg_sh.at[dst_v.at[jj]], add=True)

    plsc.subcore_barrier()
    pltpu.sync_copy(agg_sh.at[pl.ds(base, ROWS_PER_TILE)],
                    agg_out.at[c, pl.ds(base, ROWS_PER_TILE)])
    if with_deg:
      pltpu.sync_copy(deg_sh.at[pl.ds(base, ROWS_PER_TILE)],
                      deg_out.at[c, pl.ds(base, ROWS_PER_TILE)])

  return pl.kernel(
      body, out_type=out_type, mesh=mesh, scratch_types=scratch,
      compiler_params=pltpu.CompilerParams(use_tc_tiling_on_sc=False))


_BN = 1000


def _tc_first(x, wcatT, b, dout):
  din = x.shape[1]
  dh = dout // 2

  def body(x_ref, w_ref, b_ref, hs_ref, hw_ref):
    xw = jnp.dot(x_ref[...], w_ref[...], preferred_element_type=jnp.float32)
    hs_ref[...] = xw[:, :dout] + b_ref[...]
    hw_ref[0] = xw[:, dout:dout + dh]
    hw_ref[1] = xw[:, dout + dh:]

  return pl.pallas_call(
      body,
      grid=(N // _BN,),
      in_specs=[
          pl.BlockSpec((_BN, din), lambda i: (i, 0)),
          pl.BlockSpec((din, 2 * dout), lambda i: (0, 0)),
          pl.BlockSpec((1, dout), lambda i: (0, 0)),
      ],
      out_specs=[
          pl.BlockSpec((_BN, dout), lambda i: (i, 0)),
          pl.BlockSpec((NC, _BN, dh), lambda i: (0, i, 0)),
      ],
      out_shape=[
          jax.ShapeDtypeStruct((N, dout), jnp.float32),
          jax.ShapeDtypeStruct((NC, N, dh), jnp.float32),
      ],
  )(x, wcatT, b)


def _tc_mid(hs_prev, agg, deg3, wcatT, b, dout):
  din = hs_prev.shape[1]
  dh = dout // 2

  def body(hs_ref, agg_ref, deg_ref, w_ref, b_ref, hs_out, hw_out):
    a = jnp.concatenate([agg_ref[0], agg_ref[1]], axis=-1)
    inv = 1.0 / jnp.maximum(deg_ref[...], 1.0)
    h = jnp.maximum(hs_ref[...] + a * inv, 0.0)
    hw = jnp.dot(h, w_ref[...], preferred_element_type=jnp.float32)
    hs_out[...] = hw[:, :dout] + b_ref[...]
    hw_out[0] = hw[:, dout:dout + dh]
    hw_out[1] = hw[:, dout + dh:]

  return pl.pallas_call(
      body,
      grid=(N // _BN,),
      in_specs=[
          pl.BlockSpec((_BN, din), lambda i: (i, 0)),
          pl.BlockSpec((NC, _BN, din // 2), lambda i: (0, i, 0)),
          pl.BlockSpec((_BN, 1), lambda i: (i, 0)),
          pl.BlockSpec((din, 2 * dout), lambda i: (0, 0)),
          pl.BlockSpec((1, dout), lambda i: (0, 0)),
      ],
      out_specs=[
          pl.BlockSpec((_BN, dout), lambda i: (i, 0)),
          pl.BlockSpec((NC, _BN, dh), lambda i: (0, i, 0)),
      ],
      out_shape=[
          jax.ShapeDtypeStruct((N, dout), jnp.float32),
          jax.ShapeDtypeStruct((NC, N, dh), jnp.float32),
      ],
  )(hs_prev, agg, deg3, wcatT, b)


def _tc_final(hs2, agg, deg3):
  dout = hs2.shape[1]

  def body(hs_ref, agg_ref, deg_ref, out_ref):
    a = jnp.concatenate([agg_ref[0], agg_ref[1]], axis=-1)
    inv = 1.0 / jnp.maximum(deg_ref[...], 1.0)
    out_ref[...] = hs_ref[...] + a * inv

  return pl.pallas_call(
      body,
      grid=(N // _BN,),
      in_specs=[
          pl.BlockSpec((_BN, dout), lambda i: (i, 0)),
          pl.BlockSpec((NC, _BN, dout // 2), lambda i: (0, i, 0)),
          pl.BlockSpec((_BN, 1), lambda i: (i, 0)),
      ],
      out_specs=pl.BlockSpec((_BN, dout), lambda i: (i, 0)),
      out_shape=jax.ShapeDtypeStruct((N, dout), jnp.float32),
  )(hs2, agg, deg3)


@jax.jit
def kernel(x, edge_index, W_self0, W_neigh0, b0, W_self1, W_neigh1, b1,
           W_self2, W_neigh2, b2):
  src = edge_index[0]
  dst = edge_index[1]
  pad = EPAD - E
  src_flat = jnp.concatenate([src, jnp.zeros((pad,), jnp.int32)])
  src_p = jnp.stack([src_flat, src_flat + N]).reshape(
      NC, EPAD // CHUNK, CHUNK)
  dst_p = jnp.concatenate(
      [dst, jnp.full((pad,), N, jnp.int32)]).reshape(EPAD // CHUNK, CHUNK)

  w0 = jnp.concatenate([W_self0, W_neigh0], axis=0).T
  w1 = jnp.concatenate([W_self1, W_neigh1], axis=0).T
  w2 = jnp.concatenate([W_self2, W_neigh2], axis=0).T

  hs0, hw0 = _tc_first(x, w0, b0.reshape(1, -1), 128)
  agg0, deg = _make_sc_agg(64, True)(hw0.reshape(NC * N, 64), src_p, dst_p)
  deg3 = deg[0].reshape(NPAD, 1)[:N]
  hs1, hw1 = _tc_mid(hs0, agg0, deg3, w1, b1.reshape(1, -1), 128)
  agg1 = _make_sc_agg(64, False)(hw1.reshape(NC * N, 64), src_p, dst_p)[0]
  hs2, hw2 = _tc_mid(hs1, agg1, deg3, w2, b2.reshape(1, -1), 64)
  agg2 = _make_sc_agg(32, False)(hw2.reshape(NC * N, 32), src_p, dst_p)[0]
  return _tc_final(hs2, agg2, deg3)


# speedup vs baseline: 6.1682x; 1.0165x over previous
import jax
import jax.numpy as jnp
from jax import lax
from jax.experimental import pallas as pl
from jax.experimental.pallas import tpu as pltpu
from jax.experimental.pallas import tpu_sc as plsc

N = 10000
E = 320000
NC = 2
NS = 16
CHUNK = 128
NPAD = 10240
EPAD = 327680
NCHUNK = EPAD // NS // CHUNK
ROWS_PER_TILE = NPAD // NS


def _make_sc_agg(dh, with_deg):
  mesh = plsc.VectorSubcoreMesh(core_axis_name="c", subcore_axis_name="s")
  out_type = [jax.ShapeDtypeStruct((NC, NPAD, dh), jnp.float32)]
  scratch = [
      pltpu.VMEM((NCHUNK, CHUNK), jnp.int32),
      pltpu.VMEM((NCHUNK, CHUNK), jnp.int32),
      pltpu.VMEM((4, CHUNK, dh), jnp.float32),
      pltpu.VMEM((64, dh), jnp.float32),
      pltpu.VMEM_SHARED((NPAD, dh), jnp.float32),
      pltpu.SemaphoreType.DMA,
      pltpu.SemaphoreType.DMA,
      pltpu.SemaphoreType.DMA,
      pltpu.SemaphoreType.DMA,
      pltpu.SemaphoreType.DMA,
      pltpu.SemaphoreType.DMA,
      pltpu.SemaphoreType.DMA,
      pltpu.SemaphoreType.DMA,
  ]
  if with_deg:
    out_type.append(jax.ShapeDtypeStruct((NC, NPAD), jnp.float32))
    scratch += [
        pltpu.VMEM((CHUNK,), jnp.float32),
        pltpu.VMEM((ROWS_PER_TILE,), jnp.float32),
        pltpu.VMEM_SHARED((NPAD,), jnp.float32),
        pltpu.SemaphoreType.DMA,
        pltpu.SemaphoreType.DMA,
    ]

  def body(hw_hbm, src_hbm, dst_hbm, agg_out, *rest):
    if with_deg:
      (deg_out, src_v, dst_v, ring, zbuf, agg_sh, g0, g1, g2, g3,
       s0, s1, s2, s3, ones_v, zdeg, deg_sh, d0, d1) = rest
      dsem = (d0, d1)
    else:
      (src_v, dst_v, ring, zbuf, agg_sh, g0, g1, g2, g3,
       s0, s1, s2, s3) = rest
    gsem = (g0, g1, g2, g3)
    ssem = (s0, s1, s2, s3)

    c = lax.axis_index("c")
    s = lax.axis_index("s")
    base = s * ROWS_PER_TILE
    zeros16 = jnp.zeros((16,), jnp.float32)

    @pl.loop(0, 64)
    def _zero_rows(r):
      for k in range(dh // 16):
        zbuf[r, pl.ds(k * 16, 16)] = zeros16

    if with_deg:
      ones16 = jnp.ones((16,), jnp.float32)
      for k in range(CHUNK // 16):
        ones_v[pl.ds(k * 16, 16)] = ones16

      @pl.loop(0, ROWS_PER_TILE // 16)
      def _zero_deg(r):
        zdeg[pl.ds(r * 16, 16)] = zeros16

    pltpu.sync_copy(src_hbm.at[c, pl.ds(s * NCHUNK, NCHUNK)], src_v)
    pltpu.sync_copy(dst_hbm.at[pl.ds(s * NCHUNK, NCHUNK)], dst_v)
    for k in range(ROWS_PER_TILE // 64):
      pltpu.sync_copy(zbuf, agg_sh.at[pl.ds(base + k * 64, 64)])
    if with_deg:
      pltpu.sync_copy(zdeg, deg_sh.at[pl.ds(base, ROWS_PER_TILE)])
    plsc.subcore_barrier()

    def wait_gather(jj, b):
      pltpu.make_async_copy(hw_hbm.at[src_v.at[jj]], ring.at[b], gsem[b]).wait()

    def start_gather(jj, b):
      pltpu.async_copy(hw_hbm.at[src_v.at[jj]], ring.at[b], gsem[b])

    def start_scatter(jj, b):
      pltpu.async_copy(ring.at[b], agg_sh.at[dst_v.at[jj]], ssem[b], add=True)

    def wait_scatter(b):
      pltpu.make_async_copy(ring.at[b], agg_sh.at[dst_v.at[0]], ssem[b]).wait()

    def deg_step(jj, parity, first):
      if not with_deg:
        return
      if not first:
        pltpu.make_async_copy(ones_v, deg_sh.at[dst_v.at[0]],
                              dsem[parity]).wait()
      pltpu.async_copy(ones_v, deg_sh.at[dst_v.at[jj]], dsem[parity], add=True)

    start_gather(0, 0)
    start_gather(1, 1)
    for jj in (0, 1):
      wait_gather(jj, jj)
      start_scatter(jj, jj)
      deg_step(jj, jj % 2, True)
      start_gather(jj + 2, jj + 2)

    @pl.loop(0, (NCHUNK - 4) // 4)
    def _chunks(gi):
      j0 = 2 + gi * 4
      for k in range(4):
        jj = j0 + k
        b = (2 + k) % 4
        b2 = (b + 2) % 4
        wait_gather(jj, b)
        start_scatter(jj, b)
        deg_step(jj, k % 2, False)
        wait_scatter(b2)
        start_gather(jj + 2, b2)

    for jj in (NCHUNK - 2, NCHUNK - 1):
      b = jj % 4
      wait_gather(jj, b)
      start_scatter(jj, b)
      deg_step(jj, jj % 2, False)

    for b in range(4):
      wait_scatter(b)
    if with_deg:
      for p in range(2):
        pltpu.make_async_copy(ones_v, deg_sh.at[dst_v.at[0]], dsem[p]).wait()

    plsc.subcore_barrier()
    pltpu.sync_copy(agg_sh.at[pl.ds(base, ROWS_PER_TILE)],
                    agg_out.at[c, pl.ds(base, ROWS_PER_TILE)])
    if with_deg:
      pltpu.sync_copy(deg_sh.at[pl.ds(base, ROWS_PER_TILE)],
                      deg_out.at[c, pl.ds(base, ROWS_PER_TILE)])

  return pl.kernel(
      body, out_type=out_type, mesh=mesh, scratch_types=scratch,
      compiler_params=pltpu.CompilerParams(use_tc_tiling_on_sc=False))


_BN = 1000


def _tc_first(x, wcatT, b, dout):
  din = x.shape[1]
  dh = dout // 2

  def body(x_ref, w_ref, b_ref, hs_ref, hw_ref):
    xw = jnp.dot(x_ref[...], w_ref[...], preferred_element_type=jnp.float32)
    hs_ref[...] = xw[:, :dout] + b_ref[...]
    hw_ref[0] = xw[:, dout:dout + dh]
    hw_ref[1] = xw[:, dout + dh:]

  return pl.pallas_call(
      body,
      grid=(N // _BN,),
      in_specs=[
          pl.BlockSpec((_BN, din), lambda i: (i, 0)),
          pl.BlockSpec((din, 2 * dout), lambda i: (0, 0)),
          pl.BlockSpec((1, dout), lambda i: (0, 0)),
      ],
      out_specs=[
          pl.BlockSpec((_BN, dout), lambda i: (i, 0)),
          pl.BlockSpec((NC, _BN, dh), lambda i: (0, i, 0)),
      ],
      out_shape=[
          jax.ShapeDtypeStruct((N, dout), jnp.float32),
          jax.ShapeDtypeStruct((NC, N, dh), jnp.float32),
      ],
  )(x, wcatT, b)


def _tc_mid(hs_prev, agg, deg3, wcatT, b, dout):
  din = hs_prev.shape[1]
  dh = dout // 2

  def body(hs_ref, agg_ref, deg_ref, w_ref, b_ref, hs_out, hw_out):
    a = jnp.concatenate([agg_ref[0], agg_ref[1]], axis=-1)
    inv = 1.0 / jnp.maximum(deg_ref[...], 1.0)
    h = jnp.maximum(hs_ref[...] + a * inv, 0.0)
    hw = jnp.dot(h, w_ref[...], preferred_element_type=jnp.float32)
    hs_out[...] = hw[:, :dout] + b_ref[...]
    hw_out[0] = hw[:, dout:dout + dh]
    hw_out[1] = hw[:, dout + dh:]

  return pl.pallas_call(
      body,
      grid=(N // _BN,),
      in_specs=[
          pl.BlockSpec((_BN, din), lambda i: (i, 0)),
          pl.BlockSpec((NC, _BN, din // 2), lambda i: (0, i, 0)),
          pl.BlockSpec((_BN, 1), lambda i: (i, 0)),
          pl.BlockSpec((din, 2 * dout), lambda i: (0, 0)),
          pl.BlockSpec((1, dout), lambda i: (0, 0)),
      ],
      out_specs=[
          pl.BlockSpec((_BN, dout), lambda i: (i, 0)),
          pl.BlockSpec((NC, _BN, dh), lambda i: (0, i, 0)),
      ],
      out_shape=[
          jax.ShapeDtypeStruct((N, dout), jnp.float32),
          jax.ShapeDtypeStruct((NC, N, dh), jnp.float32),
      ],
  )(hs_prev, agg, deg3, wcatT, b)


def _tc_final(hs2, agg, deg3):
  dout = hs2.shape[1]

  def body(hs_ref, agg_ref, deg_ref, out_ref):
    a = jnp.concatenate([agg_ref[0], agg_ref[1]], axis=-1)
    inv = 1.0 / jnp.maximum(deg_ref[...], 1.0)
    out_ref[...] = hs_ref[...] + a * inv

  return pl.pallas_call(
      body,
      grid=(N // _BN,),
      in_specs=[
          pl.BlockSpec((_BN, dout), lambda i: (i, 0)),
          pl.BlockSpec((NC, _BN, dout // 2), lambda i: (0, i, 0)),
          pl.BlockSpec((_BN, 1), lambda i: (i, 0)),
      ],
      out_specs=pl.BlockSpec((_BN, dout), lambda i: (i, 0)),
      out_shape=jax.ShapeDtypeStruct((N, dout), jnp.float32),
  )(hs2, agg, deg3)


@jax.jit
def kernel(x, edge_index, W_self0, W_neigh0, b0, W_self1, W_neigh1, b1,
           W_self2, W_neigh2, b2):
  src = edge_index[0]
  dst = edge_index[1]
  pad = EPAD - E
  src_flat = jnp.concatenate([src, jnp.zeros((pad,), jnp.int32)])
  src_p = jnp.stack([src_flat, src_flat + N]).reshape(
      NC, EPAD // CHUNK, CHUNK)
  dst_p = jnp.concatenate(
      [dst, jnp.full((pad,), N, jnp.int32)]).reshape(EPAD // CHUNK, CHUNK)

  w0 = jnp.concatenate([W_self0, W_neigh0], axis=0).T
  w1 = jnp.concatenate([W_self1, W_neigh1], axis=0).T
  w2 = jnp.concatenate([W_self2, W_neigh2], axis=0).T

  hs0, hw0 = _tc_first(x, w0, b0.reshape(1, -1), 128)
  agg0, deg = _make_sc_agg(64, True)(hw0.reshape(NC * N, 64), src_p, dst_p)
  deg3 = deg[0].reshape(NPAD, 1)[:N]
  hs1, hw1 = _tc_mid(hs0, agg0, deg3, w1, b1.reshape(1, -1), 128)
  agg1 = _make_sc_agg(64, False)(hw1.reshape(NC * N, 64), src_p, dst_p)[0]
  hs2, hw2 = _tc_mid(hs1, agg1, deg3, w2, b2.reshape(1, -1), 64)
  agg2 = _make_sc_agg(32, False)(hw2.reshape(NC * N, 32), src_p, dst_p)[0]
  return _tc_final(hs2, agg2, deg3)


# speedup vs baseline: 6.2039x; 1.0058x over previous
import jax
import jax.numpy as jnp
from jax import lax
from jax.experimental import pallas as pl
from jax.experimental.pallas import tpu as pltpu
from jax.experimental.pallas import tpu_sc as plsc

N = 10000
E = 320000
NC = 2
NS = 16
CHUNK = 256
NPAD = 10240
EPAD = 327680
NCHUNK = EPAD // NS // CHUNK
ROWS_PER_TILE = NPAD // NS


def _make_sc_agg(dh, with_deg):
  mesh = plsc.VectorSubcoreMesh(core_axis_name="c", subcore_axis_name="s")
  out_type = [jax.ShapeDtypeStruct((NC, NPAD, dh), jnp.float32)]
  scratch = [
      pltpu.VMEM((NCHUNK, CHUNK), jnp.int32),
      pltpu.VMEM((NCHUNK, CHUNK), jnp.int32),
      pltpu.VMEM((2, CHUNK, dh), jnp.float32),
      pltpu.VMEM((16, dh), jnp.float32),
      pltpu.VMEM_SHARED((NPAD, dh), jnp.float32),
  ] + [pltpu.SemaphoreType.DMA] * 4
  if with_deg:
    out_type.append(jax.ShapeDtypeStruct((NC, NPAD), jnp.float32))
    scratch += [
        pltpu.VMEM((CHUNK,), jnp.float32),
        pltpu.VMEM((ROWS_PER_TILE,), jnp.float32),
        pltpu.VMEM_SHARED((NPAD,), jnp.float32),
        pltpu.SemaphoreType.DMA,
        pltpu.SemaphoreType.DMA,
    ]

  def body(hw_hbm, src_hbm, dst_hbm, agg_out, *rest):
    if with_deg:
      deg_out = rest[0]
      rest = rest[1:]
    src_v, dst_v, ring, zbuf, agg_sh = rest[:5]
    gsem = rest[5:7]
    ssem = rest[7:9]
    if with_deg:
      ones_v, zdeg, deg_sh, d0, d1 = rest[9:14]
      dsem = (d0, d1)

    c = lax.axis_index("c")
    s = lax.axis_index("s")
    base = s * ROWS_PER_TILE
    zeros16 = jnp.zeros((16,), jnp.float32)

    @pl.loop(0, 16)
    def _zero_rows(r):
      for k in range(dh // 16):
        zbuf[r, pl.ds(k * 16, 16)] = zeros16

    if with_deg:
      ones16 = jnp.ones((16,), jnp.float32)
      for k in range(CHUNK // 16):
        ones_v[pl.ds(k * 16, 16)] = ones16

      @pl.loop(0, ROWS_PER_TILE // 16)
      def _zero_deg(r):
        zdeg[pl.ds(r * 16, 16)] = zeros16

    pltpu.sync_copy(src_hbm.at[c, pl.ds(s * NCHUNK, NCHUNK)], src_v)
    pltpu.sync_copy(dst_hbm.at[pl.ds(s * NCHUNK, NCHUNK)], dst_v)
    @pl.loop(0, ROWS_PER_TILE // 16)
    def _zero_acc(r):
      pltpu.sync_copy(zbuf, agg_sh.at[pl.ds(base + r * 16, 16)])
    if with_deg:
      pltpu.sync_copy(zdeg, deg_sh.at[pl.ds(base, ROWS_PER_TILE)])
    plsc.subcore_barrier()

    def wait_gather(jj, b):
      pltpu.make_async_copy(hw_hbm.at[src_v.at[jj]], ring.at[b], gsem[b]).wait()

    def start_gather(jj, b):
      pltpu.async_copy(hw_hbm.at[src_v.at[jj]], ring.at[b], gsem[b])

    def start_scatter(jj, b):
      pltpu.async_copy(ring.at[b], agg_sh.at[dst_v.at[jj]], ssem[b], add=True)

    def wait_scatter(b):
      pltpu.make_async_copy(ring.at[b], agg_sh.at[dst_v.at[0]], ssem[b]).wait()

    def deg_step(jj, parity):
      if not with_deg:
        return

      @pl.when(jj >= 2)
      def _():
        pltpu.make_async_copy(ones_v, deg_sh.at[dst_v.at[0]],
                              dsem[parity]).wait()

      pltpu.async_copy(ones_v, deg_sh.at[dst_v.at[jj]], dsem[parity], add=True)

    start_gather(0, 0)
    start_gather(1, 1)

    @pl.loop(0, NCHUNK // 2 - 1)
    def _chunks(gi):
      j0 = gi * 2
      for b in range(2):
        jj = j0 + b
        wait_gather(jj, b)
        start_scatter(jj, b)
        deg_step(jj, b)
        wait_scatter(b)
        start_gather(jj + 2, b)

    for jj in range(NCHUNK - 2, NCHUNK):
      b = jj % 2
      wait_gather(jj, b)
      start_scatter(jj, b)
      deg_step(jj, b)

    for b in range(2):
      wait_scatter(b)
    if with_deg:
      for p in range(2):
        pltpu.make_async_copy(ones_v, deg_sh.at[dst_v.at[0]], dsem[p]).wait()

    plsc.subcore_barrier()
    pltpu.sync_copy(agg_sh.at[pl.ds(base, ROWS_PER_TILE)],
                    agg_out.at[c, pl.ds(base, ROWS_PER_TILE)])
    if with_deg:
      pltpu.sync_copy(deg_sh.at[pl.ds(base, ROWS_PER_TILE)],
                      deg_out.at[c, pl.ds(base, ROWS_PER_TILE)])

  return pl.kernel(
      body, out_type=out_type, mesh=mesh, scratch_types=scratch,
      compiler_params=pltpu.CompilerParams(use_tc_tiling_on_sc=False))


_BN = 1000


def _tc_first(x, wcatT, b, dout):
  din = x.shape[1]
  dh = dout // 2

  def body(x_ref, w_ref, b_ref, hs_ref, hw_ref):
    xw = jnp.dot(x_ref[...], w_ref[...], preferred_element_type=jnp.float32)
    hs_ref[...] = xw[:, :dout] + b_ref[...]
    hw_ref[0] = xw[:, dout:dout + dh]
    hw_ref[1] = xw[:, dout + dh:]

  return pl.pallas_call(
      body,
      grid=(N // _BN,),
      in_specs=[
          pl.BlockSpec((_BN, din), lambda i: (i, 0)),
          pl.BlockSpec((din, 2 * dout), lambda i: (0, 0)),
          pl.BlockSpec((1, dout), lambda i: (0, 0)),
      ],
      out_specs=[
          pl.BlockSpec((_BN, dout), lambda i: (i, 0)),
          pl.BlockSpec((NC, _BN, dh), lambda i: (0, i, 0)),
      ],
      out_shape=[
          jax.ShapeDtypeStruct((N, dout), jnp.float32),
          jax.ShapeDtypeStruct((NC, N, dh), jnp.float32),
      ],
  )(x, wcatT, b)


def _tc_mid(hs_prev, agg, deg3, wcatT, b, dout):
  din = hs_prev.shape[1]
  dh = dout // 2

  def body(hs_ref, agg_ref, deg_ref, w_ref, b_ref, hs_out, hw_out):
    a = jnp.concatenate([agg_ref[0], agg_ref[1]], axis=-1)
    inv = 1.0 / jnp.maximum(deg_ref[...], 1.0)
    h = jnp.maximum(hs_ref[...] + a * inv, 0.0)
    hw = jnp.dot(h, w_ref[...], preferred_element_type=jnp.float32)
    hs_out[...] = hw[:, :dout] + b_ref[...]
    hw_out[0] = hw[:, dout:dout + dh]
    hw_out[1] = hw[:, dout + dh:]

  return pl.pallas_call(
      body,
      grid=(N // _BN,),
      in_specs=[
          pl.BlockSpec((_BN, din), lambda i: (i, 0)),
          pl.BlockSpec((NC, _BN, din // 2), lambda i: (0, i, 0)),
          pl.BlockSpec((_BN, 1), lambda i: (i, 0)),
          pl.BlockSpec((din, 2 * dout), lambda i: (0, 0)),
          pl.BlockSpec((1, dout), lambda i: (0, 0)),
      ],
      out_specs=[
          pl.BlockSpec((_BN, dout), lambda i: (i, 0)),
          pl.BlockSpec((NC, _BN, dh), lambda i: (0, i, 0)),
      ],
      out_shape=[
          jax.ShapeDtypeStruct((N, dout), jnp.float32),
          jax.ShapeDtypeStruct((NC, N, dh), jnp.float32),
      ],
  )(hs_prev, agg, deg3, wcatT, b)


def _tc_final(hs2, agg, deg3):
  dout = hs2.shape[1]

  def body(hs_ref, agg_ref, deg_ref, out_ref):
    a = jnp.concatenate([agg_ref[0], agg_ref[1]], axis=-1)
    inv = 1.0 / jnp.maximum(deg_ref[...], 1.0)
    out_ref[...] = hs_ref[...] + a * inv

  return pl.pallas_call(
      body,
      grid=(N // _BN,),
      in_specs=[
          pl.BlockSpec((_BN, dout), lambda i: (i, 0)),
          pl.BlockSpec((NC, _BN, dout // 2), lambda i: (0, i, 0)),
          pl.BlockSpec((_BN, 1), lambda i: (i, 0)),
      ],
      out_specs=pl.BlockSpec((_BN, dout), lambda i: (i, 0)),
      out_shape=jax.ShapeDtypeStruct((N, dout), jnp.float32),
  )(hs2, agg, deg3)


@jax.jit
def kernel(x, edge_index, W_self0, W_neigh0, b0, W_self1, W_neigh1, b1,
           W_self2, W_neigh2, b2):
  src = edge_index[0]
  dst = edge_index[1]
  pad = EPAD - E
  src_flat = jnp.concatenate([src, jnp.zeros((pad,), jnp.int32)])
  src_p = jnp.stack([src_flat, src_flat + N]).reshape(
      NC, EPAD // CHUNK, CHUNK)
  dst_p = jnp.concatenate(
      [dst, jnp.full((pad,), N, jnp.int32)]).reshape(EPAD // CHUNK, CHUNK)

  w0 = jnp.concatenate([W_self0, W_neigh0], axis=0).T
  w1 = jnp.concatenate([W_self1, W_neigh1], axis=0).T
  w2 = jnp.concatenate([W_self2, W_neigh2], axis=0).T

  hs0, hw0 = _tc_first(x, w0, b0.reshape(1, -1), 128)
  agg0, deg = _make_sc_agg(64, True)(hw0.reshape(NC * N, 64), src_p, dst_p)
  deg3 = deg[0].reshape(NPAD, 1)[:N]
  hs1, hw1 = _tc_mid(hs0, agg0, deg3, w1, b1.reshape(1, -1), 128)
  agg1 = _make_sc_agg(64, False)(hw1.reshape(NC * N, 64), src_p, dst_p)[0]
  hs2, hw2 = _tc_mid(hs1, agg1, deg3, w2, b2.reshape(1, -1), 64)
  agg2 = _make_sc_agg(32, False)(hw2.reshape(NC * N, 32), src_p, dst_p)[0]
  return _tc_final(hs2, agg2, deg3)


# speedup vs baseline: 9.6814x; 1.5605x over previous
import jax
import jax.numpy as jnp
from jax import lax
from jax.experimental import pallas as pl
from jax.experimental.pallas import tpu as pltpu
from jax.experimental.pallas import tpu_sc as plsc

N = 10000
E = 320000
NC = 2
NS = 16
CHUNK = 256
NPAD = 10240
EPAD = 327680
NCHUNK = EPAD // NS // CHUNK
ROWS_PER_TILE = NPAD // NS


def _make_sc_agg(dh, with_deg):
  mesh = plsc.VectorSubcoreMesh(core_axis_name="c", subcore_axis_name="s")
  out_type = [jax.ShapeDtypeStruct((NC, NPAD, dh), jnp.bfloat16)]
  scratch = [
      pltpu.VMEM((NCHUNK, CHUNK), jnp.int32),
      pltpu.VMEM((NCHUNK, CHUNK), jnp.int32),
      pltpu.VMEM((2, CHUNK, dh), jnp.bfloat16),
      pltpu.VMEM((16, dh), jnp.bfloat16),
      pltpu.VMEM_SHARED((NPAD, dh), jnp.bfloat16),
  ] + [pltpu.SemaphoreType.DMA] * 4
  if with_deg:
    out_type.append(jax.ShapeDtypeStruct((NC, NPAD), jnp.float32))
    scratch += [
        pltpu.VMEM((CHUNK,), jnp.float32),
        pltpu.VMEM((ROWS_PER_TILE,), jnp.float32),
        pltpu.VMEM_SHARED((NPAD,), jnp.float32),
        pltpu.SemaphoreType.DMA,
        pltpu.SemaphoreType.DMA,
    ]

  def body(hw_hbm, src_hbm, dst_hbm, agg_out, *rest):
    if with_deg:
      deg_out = rest[0]
      rest = rest[1:]
    src_v, dst_v, ring, zbuf, agg_sh = rest[:5]
    gsem = rest[5:7]
    ssem = rest[7:9]
    if with_deg:
      ones_v, zdeg, deg_sh, d0, d1 = rest[9:14]
      dsem = (d0, d1)

    c = lax.axis_index("c")
    s = lax.axis_index("s")
    base = s * ROWS_PER_TILE
    zeros16 = jnp.zeros((16,), jnp.float32)
    zeros32b = jnp.zeros((32,), jnp.bfloat16)

    @pl.loop(0, 16)
    def _zero_rows(r):
      for k in range(dh // 32):
        zbuf[r, pl.ds(k * 32, 32)] = zeros32b

    if with_deg:
      ones16 = jnp.ones((16,), jnp.float32)
      for k in range(CHUNK // 16):
        ones_v[pl.ds(k * 16, 16)] = ones16

      @pl.loop(0, ROWS_PER_TILE // 16)
      def _zero_deg(r):
        zdeg[pl.ds(r * 16, 16)] = zeros16

    pltpu.sync_copy(src_hbm.at[c, pl.ds(s * NCHUNK, NCHUNK)], src_v)
    pltpu.sync_copy(dst_hbm.at[pl.ds(s * NCHUNK, NCHUNK)], dst_v)
    @pl.loop(0, ROWS_PER_TILE // 16)
    def _zero_acc(r):
      pltpu.sync_copy(zbuf, agg_sh.at[pl.ds(base + r * 16, 16)])
    if with_deg:
      pltpu.sync_copy(zdeg, deg_sh.at[pl.ds(base, ROWS_PER_TILE)])
    plsc.subcore_barrier()

    def wait_gather(jj, b):
      pltpu.make_async_copy(hw_hbm.at[src_v.at[jj]], ring.at[b], gsem[b]).wait()

    def start_gather(jj, b):
      pltpu.async_copy(hw_hbm.at[src_v.at[jj]], ring.at[b], gsem[b])

    def start_scatter(jj, b):
      pltpu.async_copy(ring.at[b], agg_sh.at[dst_v.at[jj]], ssem[b], add=True)

    def wait_scatter(b):
      pltpu.make_async_copy(ring.at[b], agg_sh.at[dst_v.at[0]], ssem[b]).wait()

    def deg_step(jj, parity):
      if not with_deg:
        return

      @pl.when(jj >= 2)
      def _():
        pltpu.make_async_copy(ones_v, deg_sh.at[dst_v.at[0]],
                              dsem[parity]).wait()

      pltpu.async_copy(ones_v, deg_sh.at[dst_v.at[jj]], dsem[parity], add=True)

    start_gather(0, 0)
    start_gather(1, 1)

    @pl.loop(0, NCHUNK // 2 - 1)
    def _chunks(gi):
      j0 = gi * 2
      for b in range(2):
        jj = j0 + b
        wait_gather(jj, b)
        start_scatter(jj, b)
        deg_step(jj, b)
        wait_scatter(b)
        start_gather(jj + 2, b)

    for jj in range(NCHUNK - 2, NCHUNK):
      b = jj % 2
      wait_gather(jj, b)
      start_scatter(jj, b)
      deg_step(jj, b)

    for b in range(2):
      wait_scatter(b)
    if with_deg:
      for p in range(2):
        pltpu.make_async_copy(ones_v, deg_sh.at[dst_v.at[0]], dsem[p]).wait()

    plsc.subcore_barrier()
    pltpu.sync_copy(agg_sh.at[pl.ds(base, ROWS_PER_TILE)],
                    agg_out.at[c, pl.ds(base, ROWS_PER_TILE)])
    if with_deg:
      pltpu.sync_copy(deg_sh.at[pl.ds(base, ROWS_PER_TILE)],
                      deg_out.at[c, pl.ds(base, ROWS_PER_TILE)])

  return pl.kernel(
      body, out_type=out_type, mesh=mesh, scratch_types=scratch,
      compiler_params=pltpu.CompilerParams(use_tc_tiling_on_sc=False))


_BN = 1000


def _tc_first(x, wcatT, b, dout):
  din = x.shape[1]
  dh = dout // 2

  def body(x_ref, w_ref, b_ref, hs_ref, hw_ref):
    xw = jnp.dot(x_ref[...], w_ref[...], preferred_element_type=jnp.float32)
    hs_ref[...] = xw[:, :dout] + b_ref[...]
    hw_ref[0] = xw[:, dout:dout + dh].astype(jnp.bfloat16)
    hw_ref[1] = xw[:, dout + dh:].astype(jnp.bfloat16)

  return pl.pallas_call(
      body,
      grid=(N // _BN,),
      in_specs=[
          pl.BlockSpec((_BN, din), lambda i: (i, 0)),
          pl.BlockSpec((din, 2 * dout), lambda i: (0, 0)),
          pl.BlockSpec((1, dout), lambda i: (0, 0)),
      ],
      out_specs=[
          pl.BlockSpec((_BN, dout), lambda i: (i, 0)),
          pl.BlockSpec((NC, _BN, dh), lambda i: (0, i, 0)),
      ],
      out_shape=[
          jax.ShapeDtypeStruct((N, dout), jnp.float32),
          jax.ShapeDtypeStruct((NC, N, dh), jnp.bfloat16),
      ],
  )(x, wcatT, b)


def _tc_mid(hs_prev, agg, deg3, wcatT, b, dout):
  din = hs_prev.shape[1]
  dh = dout // 2

  def body(hs_ref, agg_ref, deg_ref, w_ref, b_ref, hs_out, hw_out):
    a = jnp.concatenate([agg_ref[0], agg_ref[1]], axis=-1).astype(jnp.float32)
    inv = 1.0 / jnp.maximum(deg_ref[...], 1.0)
    h = jnp.maximum(hs_ref[...] + a * inv, 0.0)
    hw = jnp.dot(h, w_ref[...], preferred_element_type=jnp.float32)
    hs_out[...] = hw[:, :dout] + b_ref[...]
    hw_out[0] = hw[:, dout:dout + dh].astype(jnp.bfloat16)
    hw_out[1] = hw[:, dout + dh:].astype(jnp.bfloat16)

  return pl.pallas_call(
      body,
      grid=(N // _BN,),
      in_specs=[
          pl.BlockSpec((_BN, din), lambda i: (i, 0)),
          pl.BlockSpec((NC, _BN, din // 2), lambda i: (0, i, 0)),
          pl.BlockSpec((_BN, 1), lambda i: (i, 0)),
          pl.BlockSpec((din, 2 * dout), lambda i: (0, 0)),
          pl.BlockSpec((1, dout), lambda i: (0, 0)),
      ],
      out_specs=[
          pl.BlockSpec((_BN, dout), lambda i: (i, 0)),
          pl.BlockSpec((NC, _BN, dh), lambda i: (0, i, 0)),
      ],
      out_shape=[
          jax.ShapeDtypeStruct((N, dout), jnp.float32),
          jax.ShapeDtypeStruct((NC, N, dh), jnp.bfloat16),
      ],
  )(hs_prev, agg, deg3, wcatT, b)


def _tc_final(hs2, agg, deg3):
  dout = hs2.shape[1]

  def body(hs_ref, agg_ref, deg_ref, out_ref):
    a = jnp.concatenate([agg_ref[0], agg_ref[1]], axis=-1).astype(jnp.float32)
    inv = 1.0 / jnp.maximum(deg_ref[...], 1.0)
    out_ref[...] = hs_ref[...] + a * inv

  return pl.pallas_call(
      body,
      grid=(N // _BN,),
      in_specs=[
          pl.BlockSpec((_BN, dout), lambda i: (i, 0)),
          pl.BlockSpec((NC, _BN, dout // 2), lambda i: (0, i, 0)),
          pl.BlockSpec((_BN, 1), lambda i: (i, 0)),
      ],
      out_specs=pl.BlockSpec((_BN, dout), lambda i: (i, 0)),
      out_shape=jax.ShapeDtypeStruct((N, dout), jnp.float32),
  )(hs2, agg, deg3)


@jax.jit
def kernel(x, edge_index, W_self0, W_neigh0, b0, W_self1, W_neigh1, b1,
           W_self2, W_neigh2, b2):
  src = edge_index[0]
  dst = edge_index[1]
  pad = EPAD - E
  src_flat = jnp.concatenate([src, jnp.zeros((pad,), jnp.int32)])
  src_p = jnp.stack([src_flat, src_flat + N]).reshape(
      NC, EPAD // CHUNK, CHUNK)
  dst_p = jnp.concatenate(
      [dst, jnp.full((pad,), N, jnp.int32)]).reshape(EPAD // CHUNK, CHUNK)

  w0 = jnp.concatenate([W_self0, W_neigh0], axis=0).T
  w1 = jnp.concatenate([W_self1, W_neigh1], axis=0).T
  w2 = jnp.concatenate([W_self2, W_neigh2], axis=0).T

  hs0, hw0 = _tc_first(x, w0, b0.reshape(1, -1), 128)
  agg0, deg = _make_sc_agg(64, True)(hw0.reshape(NC * N, 64), src_p, dst_p)
  deg3 = deg[0].reshape(NPAD, 1)[:N]
  hs1, hw1 = _tc_mid(hs0, agg0, deg3, w1, b1.reshape(1, -1), 128)
  agg1 = _make_sc_agg(64, False)(hw1.reshape(NC * N, 64), src_p, dst_p)[0]
  hs2, hw2 = _tc_mid(hs1, agg1, deg3, w2, b2.reshape(1, -1), 64)
  agg2 = _make_sc_agg(32, False)(hw2.reshape(NC * N, 32), src_p, dst_p)[0]
  return _tc_final(hs2, agg2, deg3)
